# baseline (speedup 1.0000x reference)
import jax
import jax.numpy as jnp
from jax import lax
from jax.experimental import pallas as pl
from jax.experimental.pallas import tpu as pltpu
from jax.experimental.pallas import tpu_sc as plsc

S = 131072
A = 4
E = 128
L = 16
NC = 2
NS = 16
NW = NC * NS
ROWS_PER_W = S // NW
C = 128
NCHUNK = ROWS_PER_W // C


def _dna_body(inputs_hbm, emb_hbm, pos_hbm, out_hbm,
              emb_v, win_v, pos_v, out_v, sem_in, sem_win, sem_out):
    wid = lax.axis_index("s") * NC + lax.axis_index("c")
    base = wid * ROWS_PER_W

    pltpu.sync_copy(emb_hbm, emb_v)
    emb_regs = [[emb_v[a, pl.ds(L * j, L)] for j in range(E // L)]
                for a in range(A)]

    def start_in(g, slot):
        row0 = base + g * C
        pltpu.make_async_copy(pos_hbm.at[pl.ds(row0, C)],
                              pos_v.at[slot], sem_in.at[slot]).start()
        pltpu.make_async_copy(inputs_hbm.at[pl.ds(row0 * A, C * A)],
                              win_v.at[slot], sem_win.at[slot]).start()

    start_in(0, 0)
    start_in(1, 1)

    def compute_chunk(g, slot):
        pltpu.make_async_copy(pos_hbm.at[pl.ds(base, C)],
                              pos_v.at[slot], sem_in.at[slot]).wait()
        pltpu.make_async_copy(inputs_hbm.at[pl.ds(base * A, C * A)],
                              win_v.at[slot], sem_win.at[slot]).wait()

        def grp_body(q, _):
            wv = win_v[slot, pl.ds(q * 16, 16)]
            for k in range(4):
                r = q * 4 + k
                for j in range(E // L):
                    v = pos_v[slot, r, pl.ds(L * j, L)]
                    for a in range(A):
                        v = v + wv[4 * k + a] * emb_regs[a][j]
                    out_v[slot, r, pl.ds(L * j, L)] = v
            return 0

        lax.fori_loop(0, C // 4, grp_body, 0)

        row0 = base + g * C
        pltpu.make_async_copy(out_v.at[slot],
                              out_hbm.at[pl.ds(row0, C)],
                              sem_out.at[slot]).start()

    def chunk_pair(g2, _):
        g = g2 * 2
        for slot in (0, 1):
            gg = g + slot

            @pl.when(gg >= 2)
            def _():
                pltpu.make_async_copy(out_v.at[slot],
                                      out_hbm.at[pl.ds(base, C)],
                                      sem_out.at[slot]).wait()

            compute_chunk(gg, slot)

            @pl.when(gg + 2 < NCHUNK)
            def _():
                start_in(gg + 2, slot)

        return 0

    lax.fori_loop(0, NCHUNK // 2, chunk_pair, 0)

    for slot in (0, 1):
        pltpu.make_async_copy(out_v.at[slot],
                              out_hbm.at[pl.ds(base, C)],
                              sem_out.at[slot]).wait()


@jax.jit
def _dna_decode(inputs, emb_table, pos_table):
    mesh = plsc.VectorSubcoreMesh(core_axis_name="c", subcore_axis_name="s",
                                  num_cores=NC, num_subcores=NS)
    return pl.kernel(
        _dna_body,
        out_type=jax.ShapeDtypeStruct((S, E), jnp.float32),
        mesh=mesh,
        scratch_types=[
            pltpu.VMEM((A, E), jnp.float32),
            pltpu.VMEM((2, C * A), jnp.float32),
            pltpu.VMEM((2, C, E), jnp.float32),
            pltpu.VMEM((2, C, E), jnp.float32),
            pltpu.SemaphoreType.DMA((2,)),
            pltpu.SemaphoreType.DMA((2,)),
            pltpu.SemaphoreType.DMA((2,)),
        ],
    )(inputs, emb_table, pos_table)


def kernel(inputs, emb_table, pos_table):
    inputs = inputs.reshape(S * A)
    return _dna_decode(inputs, emb_table, pos_table)


# speedup vs baseline: 1.5253x; 1.5253x over previous
import jax
import jax.numpy as jnp
from jax import lax
from jax.experimental import pallas as pl
from jax.experimental.pallas import tpu as pltpu
from jax.experimental.pallas import tpu_sc as plsc

S = 131072
A = 4
E = 128
L = 16
NC = 2
NS = 16
NW = NC * NS
ROWS_PER_W = S // NW
C = 128
NCHUNK = ROWS_PER_W // C


def _dna_body(inputs_hbm, emb_hbm, pos_hbm, out_hbm,
              emb_v, win_v, pos_v, out_v, sem_in, sem_win, sem_out):
    wid = lax.axis_index("s") * NC + lax.axis_index("c")
    base = wid * ROWS_PER_W

    pltpu.sync_copy(emb_hbm, emb_v)
    emb_regs = [[emb_v[a, pl.ds(L * j, L)] for j in range(E // L)]
                for a in range(A)]

    def start_in(g, slot):
        row0 = base + g * C
        pltpu.make_async_copy(pos_hbm.at[pl.ds(row0, C)],
                              pos_v.at[slot], sem_in.at[slot]).start()
        pltpu.make_async_copy(inputs_hbm.at[pl.ds(row0 * A, C * A)],
                              win_v.at[slot], sem_win.at[slot]).start()

    start_in(0, 0)
    start_in(1, 1)

    def compute_chunk(g, slot):
        pltpu.make_async_copy(pos_hbm.at[pl.ds(base, C)],
                              pos_v.at[slot], sem_in.at[slot]).wait()
        pltpu.make_async_copy(inputs_hbm.at[pl.ds(base * A, C * A)],
                              win_v.at[slot], sem_win.at[slot]).wait()

        @plsc.parallel_loop(0, C // 4, unroll=4)
        def grp_body(q):
            wv = win_v[slot, pl.ds(q * 16, 16)]
            for k in range(4):
                r = q * 4 + k
                for j in range(E // L):
                    v0 = wv[4 * k] * emb_regs[0][j] + wv[4 * k + 1] * emb_regs[1][j]
                    v1 = wv[4 * k + 2] * emb_regs[2][j] + wv[4 * k + 3] * emb_regs[3][j]
                    out_v[slot, r, pl.ds(L * j, L)] = (
                        pos_v[slot, r, pl.ds(L * j, L)] + (v0 + v1))

        row0 = base + g * C
        pltpu.make_async_copy(out_v.at[slot],
                              out_hbm.at[pl.ds(row0, C)],
                              sem_out.at[slot]).start()

    def chunk_pair(g2, _):
        g = g2 * 2
        for slot in (0, 1):
            gg = g + slot

            @pl.when(gg >= 2)
            def _():
                pltpu.make_async_copy(out_v.at[slot],
                                      out_hbm.at[pl.ds(base, C)],
                                      sem_out.at[slot]).wait()

            compute_chunk(gg, slot)

            @pl.when(gg + 2 < NCHUNK)
            def _():
                start_in(gg + 2, slot)

        return 0

    lax.fori_loop(0, NCHUNK // 2, chunk_pair, 0)

    for slot in (0, 1):
        pltpu.make_async_copy(out_v.at[slot],
                              out_hbm.at[pl.ds(base, C)],
                              sem_out.at[slot]).wait()


@jax.jit
def _dna_decode(inputs, emb_table, pos_table):
    mesh = plsc.VectorSubcoreMesh(core_axis_name="c", subcore_axis_name="s",
                                  num_cores=NC, num_subcores=NS)
    return pl.kernel(
        _dna_body,
        out_type=jax.ShapeDtypeStruct((S, E), jnp.float32),
        mesh=mesh,
        scratch_types=[
            pltpu.VMEM((A, E), jnp.float32),
            pltpu.VMEM((2, C * A), jnp.float32),
            pltpu.VMEM((2, C, E), jnp.float32),
            pltpu.VMEM((2, C, E), jnp.float32),
            pltpu.SemaphoreType.DMA((2,)),
            pltpu.SemaphoreType.DMA((2,)),
            pltpu.SemaphoreType.DMA((2,)),
        ],
    )(inputs, emb_table, pos_table)


def kernel(inputs, emb_table, pos_table):
    inputs = inputs.reshape(S * A)
    return _dna_decode(inputs, emb_table, pos_table)
